# baseline (speedup 1.0000x reference)
import functools

import jax
import jax.numpy as jnp
from jax import lax
from jax.experimental import pallas as pl
from jax.experimental.pallas import tpu as pltpu
from jax.experimental.pallas import tpu_sc as plsc

_NMAX = 8
_LMAX = 3
_RC = 5.0
_WIDTH = 0.5
_NSPEC = 3
_MESH = 600

_BA = 32
_ROWS = _BA * _NSPEC
_NML = (_LMAX + 1) ** 2
_FEAT = _NMAX * _NML
_ACCN = _ROWS * _FEAT
_CW = 16
_CE = _CW * 16

_L_OF_M = [0, 1, 1, 1, 2, 2, 2, 2, 2, 3, 3, 3, 3, 3, 3, 3]


def _ylm(x, y, z):
    c1 = 0.4886025119029199
    sh = [jnp.full((16,), 0.28209479177387814, jnp.float32)]
    sh += [c1 * y, c1 * z, c1 * x]
    zz = z * z
    sh += [
        1.0925484305920792 * x * y,
        1.0925484305920792 * y * z,
        0.31539156525252005 * (3.0 * zz - 1.0),
        1.0925484305920792 * x * z,
        0.5462742152960396 * (x * x - y * y),
    ]
    f5 = 5.0 * zz
    sh += [
        0.5900435899266435 * y * (3.0 * x * x - y * y),
        2.890611442640554 * x * y * z,
        0.4570457994644658 * y * (f5 - 1.0),
        0.3731763325901154 * z * (f5 - 3.0),
        0.4570457994644658 * x * (f5 - 1.0),
        1.445305721320277 * z * (x * x - y * y),
        0.5900435899266435 * x * (x * x - 3.0 * y * y),
    ]
    return sh


def _cutoff(d):
    pi = 3.141592653589793
    start = _RC - _WIDTH
    u = (d - start) * (pi / _WIDTH)
    w = jnp.clip(u - 0.5 * pi, -0.5 * pi, 0.5 * pi)
    w2 = w * w
    sinw = w * (1.0 + w2 * (-1.0 / 6.0 + w2 * (1.0 / 120.0
               + w2 * (-1.0 / 5040.0 + w2 * (1.0 / 362880.0)))))
    mid = 0.5 * (1.0 - sinw)
    return jnp.where(d < start, 1.0, jnp.where(d < _RC, mid, 0.0))


def kernel(z, idx_i, idx_j, distances, direction_vectors, n_atoms,
           species2idx, spline_coeffs):
    n_at = z.shape[0]
    n_ed = idx_i.shape[0]
    nblk = (n_at + _BA - 1) // _BA

    info = plsc.get_sparse_core_info()
    n_workers = info.num_cores * info.num_subcores
    blocks_per_tile = (nblk + n_workers - 1) // n_workers

    bounds = jnp.arange(nblk + 1, dtype=jnp.int32) * _BA
    starts = jnp.searchsorted(idx_i.astype(jnp.int32), bounds,
                              side='left').astype(jnp.int32)
    n_st = ((nblk + 1 + 16 + 15) // 16) * 16
    starts_pad = jnp.zeros((n_st,), jnp.int32).at[:nblk + 1].set(starts)

    epad = _CE + 16

    def pad_e(a):
        return jnp.pad(a, (0, epad))

    ii_a = pad_e(idx_i.astype(jnp.int32))
    jj_a = pad_e(idx_j.astype(jnp.int32))
    dd_a = pad_e(distances.astype(jnp.float32))
    dx_a = pad_e(direction_vectors[:, 0].astype(jnp.float32))
    dy_a = pad_e(direction_vectors[:, 1].astype(jnp.float32))
    dz_a = pad_e(direction_vectors[:, 2].astype(jnp.float32))

    zpad = (-n_at) % 16
    z_a = jnp.pad(z.astype(jnp.int32), (0, zpad))
    n_zt = z_a.shape[0]
    s2i_a = jnp.zeros((16,), jnp.int32).at[:species2idx.shape[0]].set(
        species2idx.astype(jnp.int32))
    tab_a = spline_coeffs.astype(jnp.float32).reshape(-1)
    n_tab = tab_a.shape[0]

    mesh = plsc.VectorSubcoreMesh(core_axis_name="c", subcore_axis_name="s")

    @functools.partial(
        pl.kernel,
        mesh=mesh,
        compiler_params=pltpu.CompilerParams(needs_layout_passes=False),
        out_type=jax.ShapeDtypeStruct((nblk * _ACCN,), jnp.float32),
        scratch_types=[
            pltpu.VMEM((n_tab,), jnp.float32),
            pltpu.VMEM((n_zt,), jnp.int32),
            pltpu.VMEM((16,), jnp.int32),
            pltpu.VMEM((n_st,), jnp.int32),
            pltpu.VMEM((_ACCN,), jnp.float32),
            pltpu.VMEM((_CE,), jnp.int32),
            pltpu.VMEM((_CE,), jnp.int32),
            pltpu.VMEM((_CE,), jnp.float32),
            pltpu.VMEM((_CE,), jnp.float32),
            pltpu.VMEM((_CE,), jnp.float32),
            pltpu.VMEM((_CE,), jnp.float32),
        ],
    )
    def sc_expand(ii_h, jj_h, dd_h, dx_h, dy_h, dz_h, z_h, s2i_h, st_h,
                  tab_h, out_h,
                  tab_v, z_v, s2i_v, st_v, acc_v,
                  ii_v, jj_v, dd_v, dx_v, dy_v, dz_v):
        cid = lax.axis_index("c")
        sid = lax.axis_index("s")
        wid = sid * info.num_cores + cid

        pltpu.sync_copy(tab_h, tab_v)
        pltpu.sync_copy(z_h, z_v)
        pltpu.sync_copy(s2i_h, s2i_v)
        pltpu.sync_copy(st_h, st_v)

        dr = _RC / (_MESH - 1)
        lane = lax.iota(jnp.int32, 16)

        def per_window(k, carry):
            s, e, base_atom, eoff0 = carry
            off = k * 16
            eid = eoff0 + off + lane
            msk = (eid >= s) & (eid < e)

            iiw = ii_v[pl.ds(off, 16)]
            jjw = jj_v[pl.ds(off, 16)]
            dw = dd_v[pl.ds(off, 16)]
            xw = dx_v[pl.ds(off, 16)]
            yw = dy_v[pl.ds(off, 16)]
            zw = dz_v[pl.ds(off, 16)]

            zj = plsc.load_gather(z_v, [jnp.clip(jjw, 0, n_zt - 1)])
            spec = plsc.load_gather(s2i_v, [jnp.clip(zj, 0, 15)])
            row = ((jnp.clip(iiw - base_atom, 0, _BA - 1)) * _NSPEC
                   + jnp.clip(spec, 0, _NSPEC - 1))
            abase = row * _FEAT

            q = dw / dr
            idx = jnp.clip(q.astype(jnp.int32), 0, _MESH - 2)
            t = (dw - idx.astype(jnp.float32) * dr) / dr
            t2 = t * t
            t3 = t2 * t
            fbase = idx * _FEAT
            cut = _cutoff(dw)
            tc1 = t * cut
            tc2 = t2 * cut
            tc3 = t3 * cut

            sh = _ylm(xw, yw, zw)

            m_of_l = [[], [], [], []]
            for m in range(_NML):
                m_of_l[_L_OF_M[m]].append(m)

            for l in range(_LMAX + 1):
                for n in range(_NMAX):
                    f = l * _NMAX + n
                    c0 = plsc.load_gather(tab_v, [fbase + f])
                    c1 = plsc.load_gather(tab_v, [fbase + (32 + f)])
                    c2 = plsc.load_gather(tab_v, [fbase + (64 + f)])
                    c3 = plsc.load_gather(tab_v, [fbase + (96 + f)])
                    val = c0 * cut + c1 * tc1 + c2 * tc2 + c3 * tc3
                    for m in m_of_l[l]:
                        plsc.addupdate_scatter(
                            acc_v, [abase + (n * _NML + m)], val * sh[m],
                            mask=msk)
            return carry

        def per_chunk(c, carry):
            s, e, base_atom, w0, nw = carry
            eoff0 = (w0 + c * _CW) * 16
            pltpu.sync_copy(ii_h.at[pl.ds(eoff0, _CE)], ii_v)
            pltpu.sync_copy(jj_h.at[pl.ds(eoff0, _CE)], jj_v)
            pltpu.sync_copy(dd_h.at[pl.ds(eoff0, _CE)], dd_v)
            pltpu.sync_copy(dx_h.at[pl.ds(eoff0, _CE)], dx_v)
            pltpu.sync_copy(dy_h.at[pl.ds(eoff0, _CE)], dy_v)
            pltpu.sync_copy(dz_h.at[pl.ds(eoff0, _CE)], dz_v)
            nwin = jnp.minimum(_CW, nw - c * _CW)

            def per_window4(k4, carry4):
                for j in range(4):
                    per_window(k4 * 4 + j, carry4)
                return carry4

            lax.fori_loop(0, (nwin + 3) // 4, per_window4,
                          (s, e, base_atom, eoff0))
            return carry

        def zero_acc(k, _):
            acc_v[pl.ds(k * 16, 16)] = jnp.zeros((16,), jnp.float32)
            return 0

        def per_block(i, _):
            b = i * n_workers + wid

            @pl.when(b < nblk)
            def _():
                se = st_v[pl.ds(b, 16)]
                s = se[0]
                e = se[1]
                base_atom = b * _BA
                lax.fori_loop(0, _ACCN // 16, zero_acc, 0)

                @pl.when(e > s)
                def _():
                    w0 = s // 16
                    nw = (e - 1) // 16 - w0 + 1
                    ncc = (nw + _CW - 1) // _CW
                    lax.fori_loop(0, ncc, per_chunk,
                                  (s, e, base_atom, w0, nw))

                pltpu.sync_copy(acc_v, out_h.at[pl.ds(b * _ACCN, _ACCN)])
            return 0

        lax.fori_loop(0, blocks_per_tile, per_block, 0)

    out = sc_expand(ii_a, jj_a, dd_a, dx_a, dy_a, dz_a, z_a, s2i_a,
                    starts_pad, tab_a)
    out = out.reshape(nblk * _BA, _NSPEC, _NMAX, _NML)[:n_at]
    return out


# speedup vs baseline: 1.1095x; 1.1095x over previous
import functools

import jax
import jax.numpy as jnp
from jax import lax
from jax.experimental import pallas as pl
from jax.experimental.pallas import tpu as pltpu
from jax.experimental.pallas import tpu_sc as plsc

_NMAX = 8
_LMAX = 3
_RC = 5.0
_WIDTH = 0.5
_NSPEC = 3
_MESH = 600

_BA = 32
_ROWS = _BA * _NSPEC
_NML = (_LMAX + 1) ** 2
_FEAT = _NMAX * _NML
_ACCN = _ROWS * _FEAT
_CW = 16
_CE = _CW * 16

_L_OF_M = [0, 1, 1, 1, 2, 2, 2, 2, 2, 3, 3, 3, 3, 3, 3, 3]


def _ylm(x, y, z):
    c1 = 0.4886025119029199
    sh = [jnp.full((16,), 0.28209479177387814, jnp.float32)]
    sh += [c1 * y, c1 * z, c1 * x]
    zz = z * z
    sh += [
        1.0925484305920792 * x * y,
        1.0925484305920792 * y * z,
        0.31539156525252005 * (3.0 * zz - 1.0),
        1.0925484305920792 * x * z,
        0.5462742152960396 * (x * x - y * y),
    ]
    f5 = 5.0 * zz
    sh += [
        0.5900435899266435 * y * (3.0 * x * x - y * y),
        2.890611442640554 * x * y * z,
        0.4570457994644658 * y * (f5 - 1.0),
        0.3731763325901154 * z * (f5 - 3.0),
        0.4570457994644658 * x * (f5 - 1.0),
        1.445305721320277 * z * (x * x - y * y),
        0.5900435899266435 * x * (x * x - 3.0 * y * y),
    ]
    return sh


def _cutoff(d):
    pi = 3.141592653589793
    start = _RC - _WIDTH
    u = (d - start) * (pi / _WIDTH)
    w = jnp.clip(u - 0.5 * pi, -0.5 * pi, 0.5 * pi)
    w2 = w * w
    sinw = w * (1.0 + w2 * (-1.0 / 6.0 + w2 * (1.0 / 120.0
               + w2 * (-1.0 / 5040.0 + w2 * (1.0 / 362880.0)))))
    mid = 0.5 * (1.0 - sinw)
    return jnp.where(d < start, 1.0, jnp.where(d < _RC, mid, 0.0))


def kernel(z, idx_i, idx_j, distances, direction_vectors, n_atoms,
           species2idx, spline_coeffs):
    n_at = z.shape[0]
    n_ed = idx_i.shape[0]
    nblk = (n_at + _BA - 1) // _BA

    info = plsc.get_sparse_core_info()
    n_workers = info.num_cores * info.num_subcores
    blocks_per_tile = (nblk + n_workers - 1) // n_workers

    bounds = jnp.arange(nblk + 1, dtype=jnp.int32) * _BA
    starts = jnp.searchsorted(idx_i.astype(jnp.int32), bounds,
                              side='left').astype(jnp.int32)
    n_st = ((nblk + 1 + 16 + 15) // 16) * 16
    starts_pad = jnp.zeros((n_st,), jnp.int32).at[:nblk + 1].set(starts)

    epad = _CE + 16

    def pad_e(a):
        return jnp.pad(a, (0, epad))

    ii_a = pad_e(idx_i.astype(jnp.int32))
    jj_a = pad_e(idx_j.astype(jnp.int32))
    dd_a = pad_e(distances.astype(jnp.float32))
    dx_a = pad_e(direction_vectors[:, 0].astype(jnp.float32))
    dy_a = pad_e(direction_vectors[:, 1].astype(jnp.float32))
    dz_a = pad_e(direction_vectors[:, 2].astype(jnp.float32))

    zpad = (-n_at) % 16
    z_a = jnp.pad(z.astype(jnp.int32), (0, zpad))
    n_zt = z_a.shape[0]
    s2i_a = jnp.zeros((16,), jnp.int32).at[:species2idx.shape[0]].set(
        species2idx.astype(jnp.int32))
    tab_a = spline_coeffs.astype(jnp.float32).reshape(-1)
    n_tab = tab_a.shape[0]

    mesh = plsc.VectorSubcoreMesh(core_axis_name="c", subcore_axis_name="s")

    @functools.partial(
        pl.kernel,
        mesh=mesh,
        compiler_params=pltpu.CompilerParams(needs_layout_passes=False),
        out_type=jax.ShapeDtypeStruct((nblk * _ACCN,), jnp.float32),
        scratch_types=[
            pltpu.VMEM((n_tab,), jnp.float32),
            pltpu.VMEM((n_zt,), jnp.int32),
            pltpu.VMEM((16,), jnp.int32),
            pltpu.VMEM((n_st,), jnp.int32),
            pltpu.VMEM((_ACCN,), jnp.float32),
            pltpu.VMEM((_CE,), jnp.int32),
            pltpu.VMEM((_CE,), jnp.int32),
            pltpu.VMEM((_CE,), jnp.float32),
            pltpu.VMEM((_CE,), jnp.float32),
            pltpu.VMEM((_CE,), jnp.float32),
            pltpu.VMEM((_CE,), jnp.float32),
        ],
    )
    def sc_expand(ii_h, jj_h, dd_h, dx_h, dy_h, dz_h, z_h, s2i_h, st_h,
                  tab_h, out_h,
                  tab_v, z_v, s2i_v, st_v, acc_v,
                  ii_v, jj_v, dd_v, dx_v, dy_v, dz_v):
        cid = lax.axis_index("c")
        sid = lax.axis_index("s")
        wid = sid * info.num_cores + cid

        pltpu.sync_copy(tab_h, tab_v)
        pltpu.sync_copy(z_h, z_v)
        pltpu.sync_copy(s2i_h, s2i_v)
        pltpu.sync_copy(st_h, st_v)

        dr = _RC / (_MESH - 1)
        lane = lax.iota(jnp.int32, 16)

        def make_window_body(s, e, base_atom, eoff0):
          def per_window(k):
            off = k * 16
            eid = eoff0 + off + lane
            msk = (eid >= s) & (eid < e)

            iiw = ii_v[pl.ds(off, 16)]
            jjw = jj_v[pl.ds(off, 16)]
            dw = dd_v[pl.ds(off, 16)]
            xw = dx_v[pl.ds(off, 16)]
            yw = dy_v[pl.ds(off, 16)]
            zw = dz_v[pl.ds(off, 16)]

            zj = plsc.load_gather(z_v, [jnp.clip(jjw, 0, n_zt - 1)])
            spec = plsc.load_gather(s2i_v, [jnp.clip(zj, 0, 15)])
            row = ((jnp.clip(iiw - base_atom, 0, _BA - 1)) * _NSPEC
                   + jnp.clip(spec, 0, _NSPEC - 1))
            abase = row * _FEAT

            q = dw / dr
            idx = jnp.clip(q.astype(jnp.int32), 0, _MESH - 2)
            t = (dw - idx.astype(jnp.float32) * dr) / dr
            t2 = t * t
            t3 = t2 * t
            fbase = idx * _FEAT
            cut = _cutoff(dw)
            tc1 = t * cut
            tc2 = t2 * cut
            tc3 = t3 * cut

            sh = _ylm(xw, yw, zw)

            m_of_l = [[], [], [], []]
            for m in range(_NML):
                m_of_l[_L_OF_M[m]].append(m)

            for l in range(_LMAX + 1):
                for n in range(_NMAX):
                    f = l * _NMAX + n
                    c0 = plsc.load_gather(tab_v, [fbase + f])
                    c1 = plsc.load_gather(tab_v, [fbase + (32 + f)])
                    c2 = plsc.load_gather(tab_v, [fbase + (64 + f)])
                    c3 = plsc.load_gather(tab_v, [fbase + (96 + f)])
                    val = c0 * cut + c1 * tc1 + c2 * tc2 + c3 * tc3
                    for m in m_of_l[l]:
                        plsc.addupdate_scatter(
                            acc_v, [abase + (n * _NML + m)], val * sh[m],
                            mask=msk)
          return per_window

        def per_chunk(c, carry):
            s, e, base_atom, w0, nw = carry
            eoff0 = (w0 + c * _CW) * 16
            pltpu.sync_copy(ii_h.at[pl.ds(eoff0, _CE)], ii_v)
            pltpu.sync_copy(jj_h.at[pl.ds(eoff0, _CE)], jj_v)
            pltpu.sync_copy(dd_h.at[pl.ds(eoff0, _CE)], dd_v)
            pltpu.sync_copy(dx_h.at[pl.ds(eoff0, _CE)], dx_v)
            pltpu.sync_copy(dy_h.at[pl.ds(eoff0, _CE)], dy_v)
            pltpu.sync_copy(dz_h.at[pl.ds(eoff0, _CE)], dz_v)
            nwin = jnp.minimum(_CW, nw - c * _CW)
            plsc.parallel_loop(0, nwin, unroll=4)(
                make_window_body(s, e, base_atom, eoff0))
            return carry

        def zero_acc(k, _):
            acc_v[pl.ds(k * 16, 16)] = jnp.zeros((16,), jnp.float32)
            return 0

        def per_block(i, _):
            b = i * n_workers + wid

            @pl.when(b < nblk)
            def _():
                se = st_v[pl.ds(b, 16)]
                s = se[0]
                e = se[1]
                base_atom = b * _BA
                lax.fori_loop(0, _ACCN // 16, zero_acc, 0)

                @pl.when(e > s)
                def _():
                    w0 = s // 16
                    nw = (e - 1) // 16 - w0 + 1
                    ncc = (nw + _CW - 1) // _CW
                    lax.fori_loop(0, ncc, per_chunk,
                                  (s, e, base_atom, w0, nw))

                pltpu.sync_copy(acc_v, out_h.at[pl.ds(b * _ACCN, _ACCN)])
            return 0

        lax.fori_loop(0, blocks_per_tile, per_block, 0)

    out = sc_expand(ii_a, jj_a, dd_a, dx_a, dy_a, dz_a, z_a, s2i_a,
                    starts_pad, tab_a)
    out = out.reshape(nblk * _BA, _NSPEC, _NMAX, _NML)[:n_at]
    return out


# speedup vs baseline: 1.1353x; 1.0232x over previous
import functools

import jax
import jax.numpy as jnp
from jax import lax
from jax.experimental import pallas as pl
from jax.experimental.pallas import tpu as pltpu
from jax.experimental.pallas import tpu_sc as plsc

_NMAX = 8
_LMAX = 3
_RC = 5.0
_WIDTH = 0.5
_NSPEC = 3
_MESH = 600

_BA = 32
_ROWS = _BA * _NSPEC
_NML = (_LMAX + 1) ** 2
_FEAT = _NMAX * _NML
_ACCN = _ROWS * _FEAT
_CW = 32
_CE = _CW * 16

_L_OF_M = [0, 1, 1, 1, 2, 2, 2, 2, 2, 3, 3, 3, 3, 3, 3, 3]


def _ylm(x, y, z):
    c1 = 0.4886025119029199
    sh = [jnp.full((16,), 0.28209479177387814, jnp.float32)]
    sh += [c1 * y, c1 * z, c1 * x]
    zz = z * z
    sh += [
        1.0925484305920792 * x * y,
        1.0925484305920792 * y * z,
        0.31539156525252005 * (3.0 * zz - 1.0),
        1.0925484305920792 * x * z,
        0.5462742152960396 * (x * x - y * y),
    ]
    f5 = 5.0 * zz
    sh += [
        0.5900435899266435 * y * (3.0 * x * x - y * y),
        2.890611442640554 * x * y * z,
        0.4570457994644658 * y * (f5 - 1.0),
        0.3731763325901154 * z * (f5 - 3.0),
        0.4570457994644658 * x * (f5 - 1.0),
        1.445305721320277 * z * (x * x - y * y),
        0.5900435899266435 * x * (x * x - 3.0 * y * y),
    ]
    return sh


def _cutoff(d):
    pi = 3.141592653589793
    start = _RC - _WIDTH
    u = (d - start) * (pi / _WIDTH)
    w = jnp.clip(u - 0.5 * pi, -0.5 * pi, 0.5 * pi)
    w2 = w * w
    sinw = w * (1.0 + w2 * (-1.0 / 6.0 + w2 * (1.0 / 120.0
               + w2 * (-1.0 / 5040.0 + w2 * (1.0 / 362880.0)))))
    mid = 0.5 * (1.0 - sinw)
    return jnp.where(d < start, 1.0, jnp.where(d < _RC, mid, 0.0))


def kernel(z, idx_i, idx_j, distances, direction_vectors, n_atoms,
           species2idx, spline_coeffs):
    n_at = z.shape[0]
    n_ed = idx_i.shape[0]
    nblk = (n_at + _BA - 1) // _BA

    info = plsc.get_sparse_core_info()
    n_workers = info.num_cores * info.num_subcores
    blocks_per_tile = (nblk + n_workers - 1) // n_workers

    bounds = jnp.arange(nblk + 1, dtype=jnp.int32) * _BA
    starts = jnp.searchsorted(idx_i.astype(jnp.int32), bounds,
                              side='left').astype(jnp.int32)
    n_st = ((nblk + 1 + 16 + 15) // 16) * 16
    starts_pad = jnp.zeros((n_st,), jnp.int32).at[:nblk + 1].set(starts)

    epad = _CE + 16

    def pad_e(a):
        return jnp.pad(a, (0, epad))

    ii_a = pad_e(idx_i.astype(jnp.int32))
    jj_a = pad_e(idx_j.astype(jnp.int32))
    dd_a = pad_e(distances.astype(jnp.float32))
    dx_a = pad_e(direction_vectors[:, 0].astype(jnp.float32))
    dy_a = pad_e(direction_vectors[:, 1].astype(jnp.float32))
    dz_a = pad_e(direction_vectors[:, 2].astype(jnp.float32))

    zpad = (-n_at) % 16
    z_a = jnp.pad(z.astype(jnp.int32), (0, zpad))
    n_zt = z_a.shape[0]
    s2i_a = jnp.zeros((16,), jnp.int32).at[:species2idx.shape[0]].set(
        species2idx.astype(jnp.int32))
    tab_a = spline_coeffs.astype(jnp.float32).reshape(-1)
    n_tab = tab_a.shape[0]

    mesh = plsc.VectorSubcoreMesh(core_axis_name="c", subcore_axis_name="s")

    @functools.partial(
        pl.kernel,
        mesh=mesh,
        compiler_params=pltpu.CompilerParams(needs_layout_passes=False),
        out_type=jax.ShapeDtypeStruct((nblk * _ACCN,), jnp.float32),
        scratch_types=[
            pltpu.VMEM((n_tab,), jnp.float32),
            pltpu.VMEM((n_zt,), jnp.int32),
            pltpu.VMEM((16,), jnp.int32),
            pltpu.VMEM((n_st,), jnp.int32),
            pltpu.VMEM((_ACCN,), jnp.float32),
            pltpu.VMEM((_CE,), jnp.int32),
            pltpu.VMEM((_CE,), jnp.int32),
            pltpu.VMEM((_CE,), jnp.float32),
            pltpu.VMEM((_CE,), jnp.float32),
            pltpu.VMEM((_CE,), jnp.float32),
            pltpu.VMEM((_CE,), jnp.float32),
        ],
    )
    def sc_expand(ii_h, jj_h, dd_h, dx_h, dy_h, dz_h, z_h, s2i_h, st_h,
                  tab_h, out_h,
                  tab_v, z_v, s2i_v, st_v, acc_v,
                  ii_v, jj_v, dd_v, dx_v, dy_v, dz_v):
        cid = lax.axis_index("c")
        sid = lax.axis_index("s")
        wid = sid * info.num_cores + cid

        pltpu.sync_copy(tab_h, tab_v)
        pltpu.sync_copy(z_h, z_v)
        pltpu.sync_copy(s2i_h, s2i_v)
        pltpu.sync_copy(st_h, st_v)

        dr = _RC / (_MESH - 1)
        lane = lax.iota(jnp.int32, 16)

        def make_window_body(s, e, base_atom, eoff0):
          def per_window(k):
            off = k * 16
            eid = eoff0 + off + lane
            msk = (eid >= s) & (eid < e)

            iiw = ii_v[pl.ds(off, 16)]
            jjw = jj_v[pl.ds(off, 16)]
            dw = dd_v[pl.ds(off, 16)]
            xw = dx_v[pl.ds(off, 16)]
            yw = dy_v[pl.ds(off, 16)]
            zw = dz_v[pl.ds(off, 16)]

            zj = plsc.load_gather(z_v, [jnp.clip(jjw, 0, n_zt - 1)])
            spec = plsc.load_gather(s2i_v, [jnp.clip(zj, 0, 15)])
            row = ((jnp.clip(iiw - base_atom, 0, _BA - 1)) * _NSPEC
                   + jnp.clip(spec, 0, _NSPEC - 1))
            abase = row * _FEAT

            q = dw / dr
            idx = jnp.clip(q.astype(jnp.int32), 0, _MESH - 2)
            t = (dw - idx.astype(jnp.float32) * dr) / dr
            t2 = t * t
            t3 = t2 * t
            fbase = idx * _FEAT
            cut = _cutoff(dw)
            tc1 = t * cut
            tc2 = t2 * cut
            tc3 = t3 * cut

            sh = _ylm(xw, yw, zw)

            m_of_l = [[], [], [], []]
            for m in range(_NML):
                m_of_l[_L_OF_M[m]].append(m)

            for l in range(_LMAX + 1):
                for n in range(_NMAX):
                    f = l * _NMAX + n
                    c0 = plsc.load_gather(tab_v, [fbase + f])
                    c1 = plsc.load_gather(tab_v, [fbase + (32 + f)])
                    c2 = plsc.load_gather(tab_v, [fbase + (64 + f)])
                    c3 = plsc.load_gather(tab_v, [fbase + (96 + f)])
                    val = c0 * cut + c1 * tc1 + c2 * tc2 + c3 * tc3
                    for m in m_of_l[l]:
                        plsc.addupdate_scatter(
                            acc_v, [abase + (n * _NML + m)], val * sh[m],
                            mask=msk)
          return per_window

        def per_chunk(c, carry):
            s, e, base_atom, w0, nw = carry
            eoff0 = (w0 + c * _CW) * 16
            pltpu.sync_copy(ii_h.at[pl.ds(eoff0, _CE)], ii_v)
            pltpu.sync_copy(jj_h.at[pl.ds(eoff0, _CE)], jj_v)
            pltpu.sync_copy(dd_h.at[pl.ds(eoff0, _CE)], dd_v)
            pltpu.sync_copy(dx_h.at[pl.ds(eoff0, _CE)], dx_v)
            pltpu.sync_copy(dy_h.at[pl.ds(eoff0, _CE)], dy_v)
            pltpu.sync_copy(dz_h.at[pl.ds(eoff0, _CE)], dz_v)
            nwin = jnp.minimum(_CW, nw - c * _CW)
            plsc.parallel_loop(0, nwin, unroll=8)(
                make_window_body(s, e, base_atom, eoff0))
            return carry

        def zero_acc(k, _):
            acc_v[pl.ds(k * 16, 16)] = jnp.zeros((16,), jnp.float32)
            return 0

        def per_block(i, _):
            b = i * n_workers + wid

            @pl.when(b < nblk)
            def _():
                se = st_v[pl.ds(b, 16)]
                s = se[0]
                e = se[1]
                base_atom = b * _BA
                lax.fori_loop(0, _ACCN // 16, zero_acc, 0)

                @pl.when(e > s)
                def _():
                    w0 = s // 16
                    nw = (e - 1) // 16 - w0 + 1
                    ncc = (nw + _CW - 1) // _CW
                    lax.fori_loop(0, ncc, per_chunk,
                                  (s, e, base_atom, w0, nw))

                pltpu.sync_copy(acc_v, out_h.at[pl.ds(b * _ACCN, _ACCN)])
            return 0

        lax.fori_loop(0, blocks_per_tile, per_block, 0)

    out = sc_expand(ii_a, jj_a, dd_a, dx_a, dy_a, dz_a, z_a, s2i_a,
                    starts_pad, tab_a)
    out = out.reshape(nblk * _BA, _NSPEC, _NMAX, _NML)[:n_at]
    return out


# speedup vs baseline: 2.0540x; 1.8092x over previous
import functools

import jax
import jax.numpy as jnp
from jax import lax
from jax.experimental import pallas as pl
from jax.experimental.pallas import tpu as pltpu
from jax.experimental.pallas import tpu_sc as plsc

_NMAX = 8
_LMAX = 3
_RC = 5.0
_WIDTH = 0.5
_NSPEC = 3
_MESH = 600

_BA = 32
_ROWS = _BA * _NSPEC
_NML = (_LMAX + 1) ** 2
_FEAT = _NMAX * _NML
_PSTR = _FEAT + 1
_ACCN = _ROWS * _PSTR
_CW = 32
_CE = _CW * 16

_L_OF_M = [0, 1, 1, 1, 2, 2, 2, 2, 2, 3, 3, 3, 3, 3, 3, 3]


def _ylm(x, y, z):
    c1 = 0.4886025119029199
    sh = [jnp.full((16,), 0.28209479177387814, jnp.float32)]
    sh += [c1 * y, c1 * z, c1 * x]
    zz = z * z
    sh += [
        1.0925484305920792 * x * y,
        1.0925484305920792 * y * z,
        0.31539156525252005 * (3.0 * zz - 1.0),
        1.0925484305920792 * x * z,
        0.5462742152960396 * (x * x - y * y),
    ]
    f5 = 5.0 * zz
    sh += [
        0.5900435899266435 * y * (3.0 * x * x - y * y),
        2.890611442640554 * x * y * z,
        0.4570457994644658 * y * (f5 - 1.0),
        0.3731763325901154 * z * (f5 - 3.0),
        0.4570457994644658 * x * (f5 - 1.0),
        1.445305721320277 * z * (x * x - y * y),
        0.5900435899266435 * x * (x * x - 3.0 * y * y),
    ]
    return sh


def _cutoff(d):
    pi = 3.141592653589793
    start = _RC - _WIDTH
    u = (d - start) * (pi / _WIDTH)
    w = jnp.clip(u - 0.5 * pi, -0.5 * pi, 0.5 * pi)
    w2 = w * w
    sinw = w * (1.0 + w2 * (-1.0 / 6.0 + w2 * (1.0 / 120.0
               + w2 * (-1.0 / 5040.0 + w2 * (1.0 / 362880.0)))))
    mid = 0.5 * (1.0 - sinw)
    return jnp.where(d < start, 1.0, jnp.where(d < _RC, mid, 0.0))


def kernel(z, idx_i, idx_j, distances, direction_vectors, n_atoms,
           species2idx, spline_coeffs):
    n_at = z.shape[0]
    n_ed = idx_i.shape[0]
    nblk = (n_at + _BA - 1) // _BA

    info = plsc.get_sparse_core_info()
    n_workers = info.num_cores * info.num_subcores
    blocks_per_tile = (nblk + n_workers - 1) // n_workers

    bounds = jnp.arange(nblk + 1, dtype=jnp.int32) * _BA
    starts = jnp.searchsorted(idx_i.astype(jnp.int32), bounds,
                              side='left').astype(jnp.int32)
    n_st = ((nblk + 1 + 16 + 15) // 16) * 16
    starts_pad = jnp.zeros((n_st,), jnp.int32).at[:nblk + 1].set(starts)

    epad = _CE + 16

    def pad_e(a):
        return jnp.pad(a, (0, epad))

    ii_a = pad_e(idx_i.astype(jnp.int32))
    jj_a = pad_e(idx_j.astype(jnp.int32))
    dd_a = pad_e(distances.astype(jnp.float32))
    dx_a = pad_e(direction_vectors[:, 0].astype(jnp.float32))
    dy_a = pad_e(direction_vectors[:, 1].astype(jnp.float32))
    dz_a = pad_e(direction_vectors[:, 2].astype(jnp.float32))

    zpad = (-n_at) % 16
    z_a = jnp.pad(z.astype(jnp.int32), (0, zpad))
    n_zt = z_a.shape[0]
    s2i_a = jnp.zeros((16,), jnp.int32).at[:species2idx.shape[0]].set(
        species2idx.astype(jnp.int32))
    tab2 = jnp.zeros((_MESH, _PSTR), jnp.float32).at[:, :_FEAT].set(
        spline_coeffs.astype(jnp.float32).reshape(_MESH, _FEAT))
    tab_a = tab2.reshape(-1)
    n_tab = tab_a.shape[0]

    mesh = plsc.VectorSubcoreMesh(core_axis_name="c", subcore_axis_name="s")

    @functools.partial(
        pl.kernel,
        mesh=mesh,
        compiler_params=pltpu.CompilerParams(needs_layout_passes=False),
        out_type=jax.ShapeDtypeStruct((nblk * _ACCN,), jnp.float32),
        scratch_types=[
            pltpu.VMEM((n_tab,), jnp.float32),
            pltpu.VMEM((n_zt,), jnp.int32),
            pltpu.VMEM((16,), jnp.int32),
            pltpu.VMEM((n_st,), jnp.int32),
            pltpu.VMEM((_ACCN,), jnp.float32),
            pltpu.VMEM((_CE,), jnp.int32),
            pltpu.VMEM((_CE,), jnp.int32),
            pltpu.VMEM((_CE,), jnp.float32),
            pltpu.VMEM((_CE,), jnp.float32),
            pltpu.VMEM((_CE,), jnp.float32),
            pltpu.VMEM((_CE,), jnp.float32),
        ],
    )
    def sc_expand(ii_h, jj_h, dd_h, dx_h, dy_h, dz_h, z_h, s2i_h, st_h,
                  tab_h, out_h,
                  tab_v, z_v, s2i_v, st_v, acc_v,
                  ii_v, jj_v, dd_v, dx_v, dy_v, dz_v):
        cid = lax.axis_index("c")
        sid = lax.axis_index("s")
        wid = sid * info.num_cores + cid

        pltpu.sync_copy(tab_h, tab_v)
        pltpu.sync_copy(z_h, z_v)
        pltpu.sync_copy(s2i_h, s2i_v)
        pltpu.sync_copy(st_h, st_v)

        dr = _RC / (_MESH - 1)
        lane = lax.iota(jnp.int32, 16)

        def make_window_body(s, e, base_atom, eoff0):
          def per_window(k):
            off = k * 16
            eid = eoff0 + off + lane
            msk = (eid >= s) & (eid < e)

            iiw = ii_v[pl.ds(off, 16)]
            jjw = jj_v[pl.ds(off, 16)]
            dw = dd_v[pl.ds(off, 16)]
            xw = dx_v[pl.ds(off, 16)]
            yw = dy_v[pl.ds(off, 16)]
            zw = dz_v[pl.ds(off, 16)]

            zj = plsc.load_gather(z_v, [jnp.clip(jjw, 0, n_zt - 1)])
            spec = plsc.load_gather(s2i_v, [jnp.clip(zj, 0, 15)])
            row = ((jnp.clip(iiw - base_atom, 0, _BA - 1)) * _NSPEC
                   + jnp.clip(spec, 0, _NSPEC - 1))
            abase = row * _PSTR

            q = dw / dr
            idx = jnp.clip(q.astype(jnp.int32), 0, _MESH - 2)
            t = (dw - idx.astype(jnp.float32) * dr) / dr
            t2 = t * t
            t3 = t2 * t
            fbase = idx * _PSTR
            cut = _cutoff(dw)
            tc1 = t * cut
            tc2 = t2 * cut
            tc3 = t3 * cut

            sh = _ylm(xw, yw, zw)

            m_of_l = [[], [], [], []]
            for m in range(_NML):
                m_of_l[_L_OF_M[m]].append(m)

            for l in range(_LMAX + 1):
                for n in range(_NMAX):
                    f = l * _NMAX + n
                    c0 = plsc.load_gather(tab_v, [fbase + f])
                    c1 = plsc.load_gather(tab_v, [fbase + (32 + f)])
                    c2 = plsc.load_gather(tab_v, [fbase + (64 + f)])
                    c3 = plsc.load_gather(tab_v, [fbase + (96 + f)])
                    val = c0 * cut + c1 * tc1 + c2 * tc2 + c3 * tc3
                    for m in m_of_l[l]:
                        plsc.addupdate_scatter(
                            acc_v, [abase + (n * _NML + m)], val * sh[m],
                            mask=msk)
          return per_window

        def per_chunk(c, carry):
            s, e, base_atom, w0, nw = carry
            eoff0 = (w0 + c * _CW) * 16
            pltpu.sync_copy(ii_h.at[pl.ds(eoff0, _CE)], ii_v)
            pltpu.sync_copy(jj_h.at[pl.ds(eoff0, _CE)], jj_v)
            pltpu.sync_copy(dd_h.at[pl.ds(eoff0, _CE)], dd_v)
            pltpu.sync_copy(dx_h.at[pl.ds(eoff0, _CE)], dx_v)
            pltpu.sync_copy(dy_h.at[pl.ds(eoff0, _CE)], dy_v)
            pltpu.sync_copy(dz_h.at[pl.ds(eoff0, _CE)], dz_v)
            nwin = jnp.minimum(_CW, nw - c * _CW)
            plsc.parallel_loop(0, nwin, unroll=8)(
                make_window_body(s, e, base_atom, eoff0))
            return carry

        def zero_acc(k, _):
            acc_v[pl.ds(k * 16, 16)] = jnp.zeros((16,), jnp.float32)
            return 0

        def per_block(i, _):
            b = i * n_workers + wid

            @pl.when(b < nblk)
            def _():
                se = st_v[pl.ds(b, 16)]
                s = se[0]
                e = se[1]
                base_atom = b * _BA
                lax.fori_loop(0, _ACCN // 16, zero_acc, 0)

                @pl.when(e > s)
                def _():
                    w0 = s // 16
                    nw = (e - 1) // 16 - w0 + 1
                    ncc = (nw + _CW - 1) // _CW
                    lax.fori_loop(0, ncc, per_chunk,
                                  (s, e, base_atom, w0, nw))

                pltpu.sync_copy(acc_v, out_h.at[pl.ds(b * _ACCN, _ACCN)])
            return 0

        lax.fori_loop(0, blocks_per_tile, per_block, 0)

    out = sc_expand(ii_a, jj_a, dd_a, dx_a, dy_a, dz_a, z_a, s2i_a,
                    starts_pad, tab_a)
    out = out.reshape(nblk * _ROWS, _PSTR)[:, :_FEAT]
    out = out.reshape(nblk * _BA, _NSPEC, _NMAX, _NML)[:n_at]
    return out


# speedup vs baseline: 2.1275x; 1.0358x over previous
import functools

import jax
import jax.numpy as jnp
from jax import lax
from jax.experimental import pallas as pl
from jax.experimental.pallas import tpu as pltpu
from jax.experimental.pallas import tpu_sc as plsc

_NMAX = 8
_LMAX = 3
_RC = 5.0
_WIDTH = 0.5
_NSPEC = 3
_MESH = 600

_BA = 64
_ROWS = _BA * _NSPEC
_NML = (_LMAX + 1) ** 2
_FEAT = _NMAX * _NML
_PSTR = _FEAT + 1
_ACCN = _ROWS * _PSTR
_CW = 32
_CE = _CW * 16

_L_OF_M = [0, 1, 1, 1, 2, 2, 2, 2, 2, 3, 3, 3, 3, 3, 3, 3]


def _ylm(x, y, z):
    c1 = 0.4886025119029199
    sh = [jnp.full((16,), 0.28209479177387814, jnp.float32)]
    sh += [c1 * y, c1 * z, c1 * x]
    zz = z * z
    sh += [
        1.0925484305920792 * x * y,
        1.0925484305920792 * y * z,
        0.31539156525252005 * (3.0 * zz - 1.0),
        1.0925484305920792 * x * z,
        0.5462742152960396 * (x * x - y * y),
    ]
    f5 = 5.0 * zz
    sh += [
        0.5900435899266435 * y * (3.0 * x * x - y * y),
        2.890611442640554 * x * y * z,
        0.4570457994644658 * y * (f5 - 1.0),
        0.3731763325901154 * z * (f5 - 3.0),
        0.4570457994644658 * x * (f5 - 1.0),
        1.445305721320277 * z * (x * x - y * y),
        0.5900435899266435 * x * (x * x - 3.0 * y * y),
    ]
    return sh


def _cutoff(d):
    pi = 3.141592653589793
    start = _RC - _WIDTH
    u = (d - start) * (pi / _WIDTH)
    w = jnp.clip(u - 0.5 * pi, -0.5 * pi, 0.5 * pi)
    w2 = w * w
    sinw = w * (1.0 + w2 * (-1.0 / 6.0 + w2 * (1.0 / 120.0
               + w2 * (-1.0 / 5040.0 + w2 * (1.0 / 362880.0)))))
    mid = 0.5 * (1.0 - sinw)
    return jnp.where(d < start, 1.0, jnp.where(d < _RC, mid, 0.0))


def kernel(z, idx_i, idx_j, distances, direction_vectors, n_atoms,
           species2idx, spline_coeffs):
    n_at = z.shape[0]
    n_ed = idx_i.shape[0]
    nblk = (n_at + _BA - 1) // _BA

    info = plsc.get_sparse_core_info()
    n_workers = info.num_cores * info.num_subcores
    blocks_per_tile = (nblk + n_workers - 1) // n_workers

    bounds = jnp.arange(nblk + 1, dtype=jnp.int32) * _BA
    starts = jnp.searchsorted(idx_i.astype(jnp.int32), bounds,
                              side='left').astype(jnp.int32)
    n_st = ((nblk + 1 + 16 + 15) // 16) * 16
    starts_pad = jnp.zeros((n_st,), jnp.int32).at[:nblk + 1].set(starts)

    epad = _CE + 16

    def pad_e(a):
        return jnp.pad(a, (0, epad))

    ii_a = pad_e(idx_i.astype(jnp.int32))
    jj_a = pad_e(idx_j.astype(jnp.int32))
    dd_a = pad_e(distances.astype(jnp.float32))
    dx_a = pad_e(direction_vectors[:, 0].astype(jnp.float32))
    dy_a = pad_e(direction_vectors[:, 1].astype(jnp.float32))
    dz_a = pad_e(direction_vectors[:, 2].astype(jnp.float32))

    zpad = (-n_at) % 16
    z_a = jnp.pad(z.astype(jnp.int32), (0, zpad))
    n_zt = z_a.shape[0]
    s2i_a = jnp.zeros((16,), jnp.int32).at[:species2idx.shape[0]].set(
        species2idx.astype(jnp.int32))
    tab2 = jnp.zeros((_MESH, _PSTR), jnp.float32).at[:, :_FEAT].set(
        spline_coeffs.astype(jnp.float32).reshape(_MESH, _FEAT))
    tab_a = tab2.reshape(-1)
    n_tab = tab_a.shape[0]

    mesh = plsc.VectorSubcoreMesh(core_axis_name="c", subcore_axis_name="s")

    @functools.partial(
        pl.kernel,
        mesh=mesh,
        compiler_params=pltpu.CompilerParams(needs_layout_passes=False),
        out_type=jax.ShapeDtypeStruct((nblk * _ACCN,), jnp.float32),
        scratch_types=[
            pltpu.VMEM((n_tab,), jnp.float32),
            pltpu.VMEM((n_zt,), jnp.int32),
            pltpu.VMEM((16,), jnp.int32),
            pltpu.VMEM((n_st,), jnp.int32),
            pltpu.VMEM((_ACCN,), jnp.float32),
            pltpu.VMEM((_CE,), jnp.int32),
            pltpu.VMEM((_CE,), jnp.int32),
            pltpu.VMEM((_CE,), jnp.float32),
            pltpu.VMEM((_CE,), jnp.float32),
            pltpu.VMEM((_CE,), jnp.float32),
            pltpu.VMEM((_CE,), jnp.float32),
        ],
    )
    def sc_expand(ii_h, jj_h, dd_h, dx_h, dy_h, dz_h, z_h, s2i_h, st_h,
                  tab_h, out_h,
                  tab_v, z_v, s2i_v, st_v, acc_v,
                  ii_v, jj_v, dd_v, dx_v, dy_v, dz_v):
        cid = lax.axis_index("c")
        sid = lax.axis_index("s")
        wid = sid * info.num_cores + cid

        pltpu.sync_copy(tab_h, tab_v)
        pltpu.sync_copy(z_h, z_v)
        pltpu.sync_copy(s2i_h, s2i_v)
        pltpu.sync_copy(st_h, st_v)

        dr = _RC / (_MESH - 1)
        lane = lax.iota(jnp.int32, 16)

        def make_window_body(s, e, base_atom, eoff0):
          def per_window(k):
            off = k * 16
            eid = eoff0 + off + lane
            msk = (eid >= s) & (eid < e)

            iiw = ii_v[pl.ds(off, 16)]
            jjw = jj_v[pl.ds(off, 16)]
            dw = dd_v[pl.ds(off, 16)]
            xw = dx_v[pl.ds(off, 16)]
            yw = dy_v[pl.ds(off, 16)]
            zw = dz_v[pl.ds(off, 16)]

            zj = plsc.load_gather(z_v, [jnp.clip(jjw, 0, n_zt - 1)])
            spec = plsc.load_gather(s2i_v, [jnp.clip(zj, 0, 15)])
            row = ((jnp.clip(iiw - base_atom, 0, _BA - 1)) * _NSPEC
                   + jnp.clip(spec, 0, _NSPEC - 1))
            abase = row * _PSTR

            q = dw / dr
            idx = jnp.clip(q.astype(jnp.int32), 0, _MESH - 2)
            t = (dw - idx.astype(jnp.float32) * dr) / dr
            t2 = t * t
            t3 = t2 * t
            fbase = idx * _PSTR
            cut = _cutoff(dw)
            tc1 = t * cut
            tc2 = t2 * cut
            tc3 = t3 * cut

            sh = _ylm(xw, yw, zw)

            m_of_l = [[], [], [], []]
            for m in range(_NML):
                m_of_l[_L_OF_M[m]].append(m)

            for l in range(_LMAX + 1):
                for n in range(_NMAX):
                    f = l * _NMAX + n
                    c0 = plsc.load_gather(tab_v, [fbase + f])
                    c1 = plsc.load_gather(tab_v, [fbase + (32 + f)])
                    c2 = plsc.load_gather(tab_v, [fbase + (64 + f)])
                    c3 = plsc.load_gather(tab_v, [fbase + (96 + f)])
                    val = c0 * cut + c1 * tc1 + c2 * tc2 + c3 * tc3
                    for m in m_of_l[l]:
                        plsc.addupdate_scatter(
                            acc_v, [abase + (n * _NML + m)], val * sh[m],
                            mask=msk)
          return per_window

        def per_chunk(c, carry):
            s, e, base_atom, w0, nw = carry
            eoff0 = (w0 + c * _CW) * 16
            pltpu.sync_copy(ii_h.at[pl.ds(eoff0, _CE)], ii_v)
            pltpu.sync_copy(jj_h.at[pl.ds(eoff0, _CE)], jj_v)
            pltpu.sync_copy(dd_h.at[pl.ds(eoff0, _CE)], dd_v)
            pltpu.sync_copy(dx_h.at[pl.ds(eoff0, _CE)], dx_v)
            pltpu.sync_copy(dy_h.at[pl.ds(eoff0, _CE)], dy_v)
            pltpu.sync_copy(dz_h.at[pl.ds(eoff0, _CE)], dz_v)
            nwin = jnp.minimum(_CW, nw - c * _CW)
            plsc.parallel_loop(0, nwin, unroll=8)(
                make_window_body(s, e, base_atom, eoff0))
            return carry

        def zero_acc(k, _):
            acc_v[pl.ds(k * 16, 16)] = jnp.zeros((16,), jnp.float32)
            return 0

        def per_block(i, _):
            b = i * n_workers + wid

            @pl.when(b < nblk)
            def _():
                se = st_v[pl.ds(b, 16)]
                s = se[0]
                e = se[1]
                base_atom = b * _BA
                lax.fori_loop(0, _ACCN // 16, zero_acc, 0)

                @pl.when(e > s)
                def _():
                    w0 = s // 16
                    nw = (e - 1) // 16 - w0 + 1
                    ncc = (nw + _CW - 1) // _CW
                    lax.fori_loop(0, ncc, per_chunk,
                                  (s, e, base_atom, w0, nw))

                pltpu.sync_copy(acc_v, out_h.at[pl.ds(b * _ACCN, _ACCN)])
            return 0

        lax.fori_loop(0, blocks_per_tile, per_block, 0)

    out = sc_expand(ii_a, jj_a, dd_a, dx_a, dy_a, dz_a, z_a, s2i_a,
                    starts_pad, tab_a)
    out = out.reshape(nblk * _ROWS, _PSTR)[:, :_FEAT]
    out = out.reshape(nblk * _BA, _NSPEC, _NMAX, _NML)[:n_at]
    return out


# speedup vs baseline: 2.2246x; 1.0456x over previous
import functools

import jax
import jax.numpy as jnp
from jax import lax
from jax.experimental import pallas as pl
from jax.experimental.pallas import tpu as pltpu
from jax.experimental.pallas import tpu_sc as plsc

_NMAX = 8
_LMAX = 3
_RC = 5.0
_WIDTH = 0.5
_NSPEC = 3
_MESH = 600

_BA = 64
_ROWS = _BA * _NSPEC
_NML = (_LMAX + 1) ** 2
_FEAT = _NMAX * _NML
_PSTR = _FEAT + 1
_ACCN = _ROWS * _PSTR
_CW = 32
_CE = _CW * 16

_L_OF_M = [0, 1, 1, 1, 2, 2, 2, 2, 2, 3, 3, 3, 3, 3, 3, 3]


def _ylm(x, y, z):
    c1 = 0.4886025119029199
    sh = [jnp.full((16,), 0.28209479177387814, jnp.float32)]
    sh += [c1 * y, c1 * z, c1 * x]
    zz = z * z
    sh += [
        1.0925484305920792 * x * y,
        1.0925484305920792 * y * z,
        0.31539156525252005 * (3.0 * zz - 1.0),
        1.0925484305920792 * x * z,
        0.5462742152960396 * (x * x - y * y),
    ]
    f5 = 5.0 * zz
    sh += [
        0.5900435899266435 * y * (3.0 * x * x - y * y),
        2.890611442640554 * x * y * z,
        0.4570457994644658 * y * (f5 - 1.0),
        0.3731763325901154 * z * (f5 - 3.0),
        0.4570457994644658 * x * (f5 - 1.0),
        1.445305721320277 * z * (x * x - y * y),
        0.5900435899266435 * x * (x * x - 3.0 * y * y),
    ]
    return sh


def _cutoff(d):
    pi = 3.141592653589793
    start = _RC - _WIDTH
    u = (d - start) * (pi / _WIDTH)
    w = jnp.clip(u - 0.5 * pi, -0.5 * pi, 0.5 * pi)
    w2 = w * w
    sinw = w * (1.0 + w2 * (-1.0 / 6.0 + w2 * (1.0 / 120.0
               + w2 * (-1.0 / 5040.0 + w2 * (1.0 / 362880.0)))))
    mid = 0.5 * (1.0 - sinw)
    return jnp.where(d < start, 1.0, jnp.where(d < _RC, mid, 0.0))


def kernel(z, idx_i, idx_j, distances, direction_vectors, n_atoms,
           species2idx, spline_coeffs):
    n_at = z.shape[0]
    n_ed = idx_i.shape[0]
    nblk = (n_at + _BA - 1) // _BA

    info = plsc.get_sparse_core_info()
    n_workers = info.num_cores * info.num_subcores
    blocks_per_tile = (nblk + n_workers - 1) // n_workers

    bounds = jnp.arange(nblk + 1, dtype=jnp.int32) * _BA
    starts = jnp.searchsorted(idx_i.astype(jnp.int32), bounds,
                              side='left').astype(jnp.int32)
    n_st = ((nblk + 1 + 16 + 15) // 16) * 16
    starts_pad = jnp.zeros((n_st,), jnp.int32).at[:nblk + 1].set(starts)

    epad = _CE + 16

    def pad_e(a):
        return jnp.pad(a, (0, epad))

    ii_a = pad_e(idx_i.astype(jnp.int32))
    jj_a = pad_e(idx_j.astype(jnp.int32))
    dd_a = pad_e(distances.astype(jnp.float32))
    dx_a = pad_e(direction_vectors[:, 0].astype(jnp.float32))
    dy_a = pad_e(direction_vectors[:, 1].astype(jnp.float32))
    dz_a = pad_e(direction_vectors[:, 2].astype(jnp.float32))

    zpad = (-n_at) % 16
    z_a = jnp.pad(z.astype(jnp.int32), (0, zpad))
    n_zt = z_a.shape[0]
    s2i_a = jnp.zeros((16,), jnp.int32).at[:species2idx.shape[0]].set(
        species2idx.astype(jnp.int32))
    tab2 = jnp.zeros((_MESH, _PSTR), jnp.float32).at[:, :_FEAT].set(
        spline_coeffs.astype(jnp.float32).reshape(_MESH, _FEAT))
    tab_a = tab2.reshape(-1)
    n_tab = tab_a.shape[0]

    mesh = plsc.VectorSubcoreMesh(core_axis_name="c", subcore_axis_name="s")

    @functools.partial(
        pl.kernel,
        mesh=mesh,
        compiler_params=pltpu.CompilerParams(needs_layout_passes=False),
        out_type=jax.ShapeDtypeStruct((nblk * _ACCN,), jnp.float32),
        scratch_types=[
            pltpu.VMEM((n_tab,), jnp.float32),
            pltpu.VMEM((n_zt,), jnp.int32),
            pltpu.VMEM((16,), jnp.int32),
            pltpu.VMEM((n_st,), jnp.int32),
            pltpu.VMEM((_ACCN,), jnp.float32),
            pltpu.VMEM((_CE,), jnp.int32),
            pltpu.VMEM((_CE,), jnp.int32),
            pltpu.VMEM((_CE,), jnp.float32),
            pltpu.VMEM((_CE,), jnp.float32),
            pltpu.VMEM((_CE,), jnp.float32),
            pltpu.VMEM((_CE,), jnp.float32),
            pltpu.SemaphoreType.DMA,
        ],
    )
    def sc_expand(ii_h, jj_h, dd_h, dx_h, dy_h, dz_h, z_h, s2i_h, st_h,
                  tab_h, out_h,
                  tab_v, z_v, s2i_v, st_v, acc_v,
                  ii_v, jj_v, dd_v, dx_v, dy_v, dz_v, dsem):
        cid = lax.axis_index("c")
        sid = lax.axis_index("s")
        wid = sid * info.num_cores + cid

        pltpu.sync_copy(tab_h, tab_v)
        pltpu.sync_copy(z_h, z_v)
        pltpu.sync_copy(s2i_h, s2i_v)
        pltpu.sync_copy(st_h, st_v)

        dr = _RC / (_MESH - 1)
        lane = lax.iota(jnp.int32, 16)

        def make_window_body(s, e, base_atom, eoff0):
          def per_window(k):
            off = k * 16
            eid = eoff0 + off + lane
            msk = (eid >= s) & (eid < e)

            iiw = ii_v[pl.ds(off, 16)]
            jjw = jj_v[pl.ds(off, 16)]
            dw = dd_v[pl.ds(off, 16)]
            xw = dx_v[pl.ds(off, 16)]
            yw = dy_v[pl.ds(off, 16)]
            zw = dz_v[pl.ds(off, 16)]

            zj = plsc.load_gather(z_v, [jnp.clip(jjw, 0, n_zt - 1)])
            spec = plsc.load_gather(s2i_v, [jnp.clip(zj, 0, 15)])
            row = ((jnp.clip(iiw - base_atom, 0, _BA - 1)) * _NSPEC
                   + jnp.clip(spec, 0, _NSPEC - 1))
            abase = row * _PSTR

            q = dw / dr
            idx = jnp.clip(q.astype(jnp.int32), 0, _MESH - 2)
            t = (dw - idx.astype(jnp.float32) * dr) / dr
            t2 = t * t
            t3 = t2 * t
            fbase = idx * _PSTR
            cut = _cutoff(dw)
            tc1 = t * cut
            tc2 = t2 * cut
            tc3 = t3 * cut

            sh = _ylm(xw, yw, zw)

            m_of_l = [[], [], [], []]
            for m in range(_NML):
                m_of_l[_L_OF_M[m]].append(m)

            for l in range(_LMAX + 1):
                for n in range(_NMAX):
                    f = l * _NMAX + n
                    c0 = plsc.load_gather(tab_v, [fbase + f])
                    c1 = plsc.load_gather(tab_v, [fbase + (32 + f)])
                    c2 = plsc.load_gather(tab_v, [fbase + (64 + f)])
                    c3 = plsc.load_gather(tab_v, [fbase + (96 + f)])
                    val = c0 * cut + c1 * tc1 + c2 * tc2 + c3 * tc3
                    for m in m_of_l[l]:
                        plsc.addupdate_scatter(
                            acc_v, [abase + (n * _NML + m)], val * sh[m],
                            mask=msk)
          return per_window

        def per_chunk(c, carry):
            s, e, base_atom, w0, nw = carry
            eoff0 = (w0 + c * _CW) * 16
            cps = [pltpu.async_copy(ii_h.at[pl.ds(eoff0, _CE)], ii_v, dsem),
                   pltpu.async_copy(jj_h.at[pl.ds(eoff0, _CE)], jj_v, dsem),
                   pltpu.async_copy(dd_h.at[pl.ds(eoff0, _CE)], dd_v, dsem),
                   pltpu.async_copy(dx_h.at[pl.ds(eoff0, _CE)], dx_v, dsem),
                   pltpu.async_copy(dy_h.at[pl.ds(eoff0, _CE)], dy_v, dsem),
                   pltpu.async_copy(dz_h.at[pl.ds(eoff0, _CE)], dz_v, dsem)]
            for cp in cps:
                cp.wait()
            nwin = jnp.minimum(_CW, nw - c * _CW)
            plsc.parallel_loop(0, nwin, unroll=8)(
                make_window_body(s, e, base_atom, eoff0))
            return carry

        def zero_acc(k, _):
            acc_v[pl.ds(k * 16, 16)] = jnp.zeros((16,), jnp.float32)
            return 0

        def per_block(i, _):
            b = i * n_workers + wid

            @pl.when(b < nblk)
            def _():
                se = st_v[pl.ds(b, 16)]
                s = se[0]
                e = se[1]
                base_atom = b * _BA
                lax.fori_loop(0, _ACCN // 16, zero_acc, 0)

                @pl.when(e > s)
                def _():
                    w0 = s // 16
                    nw = (e - 1) // 16 - w0 + 1
                    ncc = (nw + _CW - 1) // _CW
                    lax.fori_loop(0, ncc, per_chunk,
                                  (s, e, base_atom, w0, nw))

                pltpu.sync_copy(acc_v, out_h.at[pl.ds(b * _ACCN, _ACCN)])
            return 0

        lax.fori_loop(0, blocks_per_tile, per_block, 0)

    out = sc_expand(ii_a, jj_a, dd_a, dx_a, dy_a, dz_a, z_a, s2i_a,
                    starts_pad, tab_a)
    out = out.reshape(nblk * _ROWS, _PSTR)[:, :_FEAT]
    out = out.reshape(nblk * _BA, _NSPEC, _NMAX, _NML)[:n_at]
    return out
